# baseline (speedup 1.0000x reference)
import functools
import math

import jax
import jax.numpy as jnp
from jax import lax
from jax.experimental import pallas as pl
from jax.experimental.pallas import tpu as pltpu
from jax.experimental.pallas import tpu_sc as plsc

D_MODEL = 64
SCALE = math.sqrt(D_MODEL)

_info = plsc.get_sparse_core_info()
_NC, _NS, _L = _info.num_cores, _info.num_subcores, _info.num_lanes
_NW = _NC * _NS


@functools.partial(jax.jit, static_argnames=("b_total", "d"))
def _emb_lookup(x_flat3, weight, b_total, d):
    b_per_w = b_total // _NW
    ch = 128
    n_ch = b_per_w // ch
    mesh = plsc.VectorSubcoreMesh(core_axis_name="c", subcore_axis_name="s")

    @functools.partial(
        pl.kernel,
        mesh=mesh,
        out_type=jax.ShapeDtypeStruct((b_total, d), jnp.float32),
        scratch_types=[
            pltpu.VMEM((n_ch, ch), jnp.int32),
            pltpu.VMEM((b_per_w, d), jnp.float32),
            pltpu.SemaphoreType.DMA,
        ],
        compiler_params=pltpu.CompilerParams(use_tc_tiling_on_sc=False),
    )
    def k(idx_hbm, table_hbm, out_hbm, idx_v, rows_v, sem):
        wid = lax.axis_index("s") * _NC + lax.axis_index("c")
        base = wid * b_per_w
        pltpu.sync_copy(idx_hbm.at[wid], idx_v)
        copies = []
        for j in range(n_ch):
            copies.append(
                pltpu.async_copy(
                    table_hbm.at[idx_v.at[j]],
                    rows_v.at[pl.ds(j * ch, ch)],
                    sem,
                )
            )
        for c in copies:
            c.wait()

        def scale_row(i, carry):
            for kk in range(d // _L):
                sl = pl.ds(kk * _L, _L)
                rows_v[i, sl] = rows_v[i, sl] * SCALE
            return carry

        lax.fori_loop(0, b_per_w, scale_row, 0)
        pltpu.sync_copy(rows_v, out_hbm.at[pl.ds(base, b_per_w)])

    return k(x_flat3, weight)


def kernel(x, weight):
    b0, b1 = x.shape
    b_total = b0 * b1
    b_per_w = b_total // _NW
    ch = 128
    x3 = x.astype(jnp.int32).reshape(_NW, b_per_w // ch, ch)
    out = _emb_lookup(x3, weight, b_total, D_MODEL)
    return out.reshape(b0, b1, D_MODEL)


# speedup vs baseline: 2.1321x; 2.1321x over previous
import functools
import math

import jax
import jax.numpy as jnp
from jax import lax
from jax.experimental import pallas as pl
from jax.experimental.pallas import tpu as pltpu
from jax.experimental.pallas import tpu_sc as plsc

D_MODEL = 64
SCALE = math.sqrt(D_MODEL)

_info = plsc.get_sparse_core_info()
_NC, _NS, _L = _info.num_cores, _info.num_subcores, _info.num_lanes
_NW = _NC * _NS


@functools.partial(jax.jit, static_argnames=("b_total", "d"))
def _emb_lookup(x3, table3, b_total, d):
    n_groups, rows_per_group, _ = table3.shape
    b_per_w = b_total // _NW
    nbuf = _L
    ch = 64
    n_ch = b_per_w // ch
    mesh = plsc.VectorSubcoreMesh(core_axis_name="c", subcore_axis_name="s")

    @functools.partial(
        pl.kernel,
        mesh=mesh,
        out_type=jax.ShapeDtypeStruct((b_total, d), jnp.float32),
        scratch_types=[
            pltpu.VMEM((rows_per_group, ch), jnp.int32),
            pltpu.VMEM((nbuf, rows_per_group, d), jnp.float32),
            pltpu.VMEM((ch, d), jnp.float32),
            pltpu.SemaphoreType.DMA,
        ],
    )
    def k(idx_hbm, tab_hbm, out_hbm, idx_v, buf_v, stage_v, sem):
        wid = lax.axis_index("s") * _NC + lax.axis_index("c")
        base = wid * b_per_w
        pltpu.sync_copy(idx_hbm.at[wid], idx_v)

        def chunk_body(c, carry):
            def batch_body(b, carry2):
                idxvec = idx_v[c, pl.ds(b * nbuf, nbuf)]
                gvec = idxvec >> 3
                offvec = idxvec & 7
                for s in range(nbuf):
                    pltpu.async_copy(tab_hbm.at[gvec[s]], buf_v.at[s], sem)
                for s in range(nbuf):
                    pltpu.make_async_copy(tab_hbm.at[0], buf_v.at[s], sem).wait()
                for s in range(nbuf):
                    r = b * nbuf + s
                    for kk in range(d // _L):
                        sl = pl.ds(kk * _L, _L)
                        stage_v[r, sl] = buf_v[s, offvec[s], sl] * SCALE
                return carry2

            lax.fori_loop(0, ch // nbuf, batch_body, 0)
            pltpu.sync_copy(stage_v, out_hbm.at[pl.ds(base + c * ch, ch)])
            return carry

        lax.fori_loop(0, n_ch, chunk_body, 0)

    return k(x3, table3)


def kernel(x, weight):
    b0, b1 = x.shape
    b_total = b0 * b1
    b_per_w = b_total // _NW
    x3 = x.astype(jnp.int32).reshape(_NW, 8, b_per_w // 8)
    table3 = weight.reshape(weight.shape[0] // 8, 8, D_MODEL)
    out = _emb_lookup(x3, table3, b_total, D_MODEL)
    return out.reshape(b0, b1, D_MODEL)
